# baseline (speedup 1.0000x reference)
import functools

import jax
import jax.numpy as jnp
from jax import lax
from jax.experimental import pallas as pl
from jax.experimental.pallas import tpu as pltpu
from jax.experimental.pallas import tpu_sc as plsc

_NUM_CORES = 2
_NUM_SUBCORES = 16
_NUM_WORKERS = _NUM_CORES * _NUM_SUBCORES
_LANES = 16

_BATCH = 4096
_SEQ = 200
_EMBED = 64
_N = _BATCH * _SEQ
_CHUNK = 128
_CHUNKS_PER_WORKER = _N // (_NUM_WORKERS * _CHUNK)
_VECS_PER_ROW = _EMBED // _LANES


def _sc_body(idx_hbm, pe_hbm, table_hbm, out_hbm, pe_v, idx_v, rows_v, sem):
  wid = lax.axis_index("s") * _NUM_CORES + lax.axis_index("c")

  pltpu.sync_copy(pe_hbm, pe_v)

  base_chunk = wid * _CHUNKS_PER_WORKER

  def chunk_body(ci, _):
    flat = (base_chunk + ci) * _CHUNK
    pltpu.sync_copy(idx_hbm.at[pl.ds(flat, _CHUNK)], idx_v)
    pltpu.async_copy(table_hbm.at[idx_v], rows_v, sem).wait()

    p0 = lax.rem(flat, _SEQ)

    def row_body(r, _):
      p = p0 + r
      p = lax.select(p >= _SEQ, p - _SEQ, p)
      for j in range(_VECS_PER_ROW):
        sl = pl.ds(j * _LANES, _LANES)
        rows_v[r, sl] = rows_v[r, sl] + pe_v[p, sl]
      return 0

    lax.fori_loop(0, _CHUNK, row_body, 0)
    pltpu.sync_copy(rows_v, out_hbm.at[pl.ds(flat, _CHUNK)])
    return 0

  lax.fori_loop(0, _CHUNKS_PER_WORKER, chunk_body, 0)


@jax.jit
def _pe_lookup(idx, table, pe):
  mesh = plsc.VectorSubcoreMesh(core_axis_name="c", subcore_axis_name="s")
  return pl.kernel(
      _sc_body,
      out_type=jax.ShapeDtypeStruct((_N, _EMBED), jnp.float32),
      mesh=mesh,
      scratch_types=[
          pltpu.VMEM((_SEQ, _EMBED), jnp.float32),
          pltpu.VMEM((_CHUNK,), jnp.int32),
          pltpu.VMEM((_CHUNK, _EMBED), jnp.float32),
          pltpu.SemaphoreType.DMA,
      ],
      compiler_params=pltpu.CompilerParams(use_tc_tiling_on_sc=False),
  )(idx, pe, table)


def kernel(x, table, pe):
  idx = x.reshape(-1).astype(jnp.int32)
  out = _pe_lookup(idx, table, pe)
  return out.reshape(_BATCH, _SEQ, _EMBED)


# speedup vs baseline: 1.1622x; 1.1622x over previous
import jax
import jax.numpy as jnp
from jax import lax
from jax.experimental import pallas as pl
from jax.experimental.pallas import tpu as pltpu
from jax.experimental.pallas import tpu_sc as plsc

_NUM_CORES = 2
_NUM_SUBCORES = 16
_NUM_WORKERS = _NUM_CORES * _NUM_SUBCORES

_BATCH = 4096
_SEQ = 200
_EMBED = 64
_N = _BATCH * _SEQ
_CHUNK = 128
_CPW = _N // (_NUM_WORKERS * _CHUNK)
_NB = 8
_DEPTH = 4


def _sc_body(idx_hbm, pe_hbm, table_hbm, out_hbm,
             idx_v, rows_v, sem_p, sem_g, sem_s):
  wid = lax.axis_index("s") * _NUM_CORES + lax.axis_index("c")
  base = wid * _CPW

  pltpu.sync_copy(idx_hbm.at[pl.ds(base, _CPW)], idx_v)

  def seed_pe(c):
    s = lax.rem(c, _NB)
    p0 = lax.rem(c * _CHUNK, _SEQ)
    pltpu.async_copy(pe_hbm.at[pl.ds(p0, _CHUNK)], rows_v.at[s], sem_p)

  def drain_seed():
    pltpu.make_async_copy(
        pe_hbm.at[pl.ds(0, _CHUNK)], rows_v.at[0], sem_p).wait()

  def drain_store():
    pltpu.make_async_copy(
        rows_v.at[0], out_hbm.at[pl.ds(0, _CHUNK)], sem_s).wait()

  def drain_gather_and_store(c):
    s = lax.rem(c, _NB)
    pltpu.make_async_copy(
        table_hbm.at[idx_v.at[c]], rows_v.at[s], sem_g).wait()
    pltpu.async_copy(
        rows_v.at[s], out_hbm.at[pl.ds((base + c) * _CHUNK, _CHUNK)], sem_s)

  seed_pe(0)

  def chunk_body(c, _):
    cn = c + 1

    @pl.when(cn < _CPW)
    def _():
      @pl.when(cn >= _NB)
      def _():
        drain_store()
      seed_pe(cn)

    drain_seed()
    s = lax.rem(c, _NB)
    pltpu.async_copy(table_hbm.at[idx_v.at[c]], rows_v.at[s], sem_g, add=True)

    @pl.when(c >= _DEPTH)
    def _():
      drain_gather_and_store(c - _DEPTH)

    return 0

  lax.fori_loop(0, _CPW, chunk_body, 0)

  for k in range(_CPW - _DEPTH, _CPW):
    drain_gather_and_store(k)
  for _ in range(_NB):
    drain_store()


@jax.jit
def _pe_lookup(idx, table, pe_ext):
  mesh = plsc.VectorSubcoreMesh(core_axis_name="c", subcore_axis_name="s")
  return pl.kernel(
      _sc_body,
      out_type=jax.ShapeDtypeStruct((_N, _EMBED), jnp.float32),
      mesh=mesh,
      scratch_types=[
          pltpu.VMEM((_CPW, _CHUNK), jnp.int32),
          pltpu.VMEM((_NB, _CHUNK, _EMBED), jnp.float32),
          pltpu.SemaphoreType.DMA,
          pltpu.SemaphoreType.DMA,
          pltpu.SemaphoreType.DMA,
      ],
      compiler_params=pltpu.CompilerParams(use_tc_tiling_on_sc=False),
  )(idx, pe_ext, table)


def kernel(x, table, pe):
  idx = x.reshape(_N // _CHUNK, _CHUNK).astype(jnp.int32)
  pe_ext = jnp.concatenate([pe, pe[:_CHUNK]], axis=0)
  out = _pe_lookup(idx, table, pe_ext)
  return out.reshape(_BATCH, _SEQ, _EMBED)


# speedup vs baseline: 1.2341x; 1.0619x over previous
import jax
import jax.numpy as jnp
from jax import lax
from jax.experimental import pallas as pl
from jax.experimental.pallas import tpu as pltpu
from jax.experimental.pallas import tpu_sc as plsc

_NUM_CORES = 2
_NUM_SUBCORES = 16
_NUM_WORKERS = _NUM_CORES * _NUM_SUBCORES

_BATCH = 4096
_SEQ = 200
_EMBED = 64
_ROWS_PW = _BATCH // _NUM_WORKERS
_C0 = 128
_C1 = _SEQ - _C0
_NB = 8
_LAG = 2


def _sc_body(x_hbm, pe_hbm, table_hbm, out_hbm, idx_v, rows_v, sem_p, sem_g,
             sem_s):
  wid = lax.axis_index("s") * _NUM_CORES + lax.axis_index("c")
  row0 = wid * _ROWS_PW

  pltpu.sync_copy(x_hbm.at[pl.ds(row0, _ROWS_PW)], idx_v)

  def slot(c):
    return lax.rem(c, _NB)

  def seed(c, half):
    if half == 0:
      pltpu.async_copy(pe_hbm.at[pl.ds(0, _C0)],
                       rows_v.at[slot(c), pl.ds(0, _C0)], sem_p)
    else:
      pltpu.async_copy(pe_hbm.at[pl.ds(_C0, _C1)],
                       rows_v.at[slot(c), pl.ds(0, _C1)], sem_p)

  def drain_seed(half):
    n = _C0 if half == 0 else _C1
    pltpu.make_async_copy(pe_hbm.at[pl.ds(0, n)],
                          rows_v.at[0, pl.ds(0, n)], sem_p).wait()

  def gather(c, r, half):
    n, p0 = (_C0, 0) if half == 0 else (_C1, _C0)
    pltpu.async_copy(table_hbm.at[idx_v.at[r, pl.ds(p0, n)]],
                     rows_v.at[slot(c), pl.ds(0, n)], sem_g, add=True)

  def drain_gather_and_store(c, r, half):
    n, p0 = (_C0, 0) if half == 0 else (_C1, _C0)
    pltpu.make_async_copy(table_hbm.at[idx_v.at[r, pl.ds(p0, n)]],
                          rows_v.at[slot(c), pl.ds(0, n)], sem_g).wait()
    pltpu.async_copy(rows_v.at[slot(c), pl.ds(0, n)],
                     out_hbm.at[row0 + r, pl.ds(p0, n)], sem_s)

  def drain_store(half):
    n = _C0 if half == 0 else _C1
    pltpu.make_async_copy(rows_v.at[0, pl.ds(0, n)],
                          out_hbm.at[0, pl.ds(0, n)], sem_s).wait()

  seed(0, 0)
  seed(1, 1)

  def body(r, _):
    rn = r + 1

    @pl.when(rn < _ROWS_PW)
    def _():
      @pl.when(rn >= _NB // 2)
      def _():
        drain_store(0)
      seed(2 * rn, 0)

      @pl.when(rn >= _NB // 2)
      def _():
        drain_store(1)
      seed(2 * rn + 1, 1)

    drain_seed(0)
    gather(2 * r, r, 0)
    drain_seed(1)
    gather(2 * r + 1, r, 1)

    @pl.when(r >= _LAG)
    def _():
      rd = r - _LAG
      drain_gather_and_store(2 * rd, rd, 0)
      drain_gather_and_store(2 * rd + 1, rd, 1)

    return 0

  lax.fori_loop(0, _ROWS_PW, body, 0)

  for rd in range(_ROWS_PW - _LAG, _ROWS_PW):
    drain_gather_and_store(2 * rd, rd, 0)
    drain_gather_and_store(2 * rd + 1, rd, 1)
  for _ in range(_NB // 2):
    drain_store(0)
    drain_store(1)


@jax.jit
def _pe_lookup(x, table, pe):
  mesh = plsc.VectorSubcoreMesh(core_axis_name="c", subcore_axis_name="s")
  return pl.kernel(
      _sc_body,
      out_type=jax.ShapeDtypeStruct((_BATCH, _SEQ, _EMBED), jnp.float32),
      mesh=mesh,
      scratch_types=[
          pltpu.VMEM((_ROWS_PW, _SEQ), jnp.int32),
          pltpu.VMEM((_NB, _C0, _EMBED), jnp.float32),
          pltpu.SemaphoreType.DMA,
          pltpu.SemaphoreType.DMA,
          pltpu.SemaphoreType.DMA,
      ],
      compiler_params=pltpu.CompilerParams(use_tc_tiling_on_sc=False),
  )(x, pe, table)


def kernel(x, table, pe):
  return _pe_lookup(x.astype(jnp.int32), table, pe)
